# baseline (speedup 1.0000x reference)
import functools

import jax
import jax.numpy as jnp
from jax import lax
from jax.experimental import pallas as pl
from jax.experimental.pallas import tpu as pltpu
from jax.experimental.pallas import tpu_sc as plsc

B = 16384
D = 32
NUM_CORES = 2
NUM_SUBCORES = 16
NW = NUM_CORES * NUM_SUBCORES
BPW = B // NW
CHUNK = 4
NSETS = 2
NGROUPS = BPW // 16


@functools.partial(
    pl.kernel,
    out_type=jax.ShapeDtypeStruct((B,), jnp.float32),
    mesh=plsc.VectorSubcoreMesh(core_axis_name="c", subcore_axis_name="s"),
    scratch_types=[
        pltpu.VMEM((BPW + 16,), jnp.int32),
        pltpu.VMEM((BPW + 16,), jnp.int32),
        pltpu.VMEM((NSETS, CHUNK, D, 128), jnp.float32),
        pltpu.VMEM((NSETS, CHUNK, D, 128), jnp.float32),
        pltpu.VMEM((BPW,), jnp.float32),
        pltpu.SemaphoreType.DMA,
        pltpu.SemaphoreType.DMA,
    ],
    compiler_params=pltpu.CompilerParams(needs_layout_passes=False),
)
def _sc_dot(uidx_hbm, iidx_hbm, utab_t_hbm, itab_t_hbm, out_hbm,
            uidx_v, iidx_v, ublk_v, iblk_v, out_v, sem0, sem1):
    wid = lax.axis_index("s") * NUM_CORES + lax.axis_index("c")
    base = wid * BPW
    sems = (sem0, sem1)

    pltpu.sync_copy(uidx_hbm.at[pl.ds(base, BPW)], uidx_v.at[pl.ds(0, BPW)])
    pltpu.sync_copy(iidx_hbm.at[pl.ds(base, BPW)], iidx_v.at[pl.ds(0, BPW)])

    lanes16 = lax.iota(jnp.int32, 16)
    d_lo = lanes16
    d_hi = lanes16 + 16

    def issue(uvec, ivec, k0, st):
        for i in range(CHUNK):
            r0u = pl.multiple_of((uvec[k0 + i] // 128) * 128, 128)
            r0i = pl.multiple_of((ivec[k0 + i] // 128) * 128, 128)
            for g in range(4):
                pltpu.async_copy(
                    utab_t_hbm.at[pl.ds(8 * g, 8), pl.ds(r0u, 128)],
                    ublk_v.at[st, i, pl.ds(8 * g, 8)], sems[st])
                pltpu.async_copy(
                    itab_t_hbm.at[pl.ds(8 * g, 8), pl.ds(r0i, 128)],
                    iblk_v.at[st, i, pl.ds(8 * g, 8)], sems[st])

    def drain(st):
        for i in range(CHUNK):
            for g in range(4):
                pltpu.make_async_copy(
                    utab_t_hbm.at[pl.ds(8 * g, 8), pl.ds(0, 128)],
                    ublk_v.at[st, i, pl.ds(8 * g, 8)], sems[st]).wait()
                pltpu.make_async_copy(
                    itab_t_hbm.at[pl.ds(8 * g, 8), pl.ds(0, 128)],
                    iblk_v.at[st, i, pl.ds(8 * g, 8)], sems[st]).wait()

    uvec0 = uidx_v[pl.ds(0, 16)]
    ivec0 = iidx_v[pl.ds(0, 16)]
    issue(uvec0, ivec0, 0, 0)
    issue(uvec0, ivec0, 4, 1)

    def group_body(q, carry):
        g0 = q * 16
        uvec = uidx_v[pl.ds(g0, 16)]
        ivec = iidx_v[pl.ds(g0, 16)]
        uvec_n = uidx_v[pl.ds(g0 + 16, 16)]
        ivec_n = iidx_v[pl.ds(g0 + 16, 16)]
        res = jnp.zeros((16,), jnp.float32)
        for j in range(4):
            st = j % NSETS
            drain(st)
            for i in range(CHUNK):
                k = 4 * j + i
                lu = jnp.full((16,), uvec[k] % 128, jnp.int32)
                li = jnp.full((16,), ivec[k] % 128, jnp.int32)
                st_splat = jnp.full((16,), st, jnp.int32)
                i_splat = jnp.full((16,), i, jnp.int32)
                u1 = plsc.load_gather(ublk_v, [st_splat, i_splat, d_lo, lu])
                u2 = plsc.load_gather(ublk_v, [st_splat, i_splat, d_hi, lu])
                v1 = plsc.load_gather(iblk_v, [st_splat, i_splat, d_lo, li])
                v2 = plsc.load_gather(iblk_v, [st_splat, i_splat, d_hi, li])
                dot = jnp.sum(u1 * v1 + u2 * v2)
                res = jnp.where(lanes16 == k, dot, res)
            if j < 2:
                issue(uvec, ivec, 4 * (j + 2), st)
            else:
                @pl.when(q < NGROUPS - 1)
                def _():
                    issue(uvec_n, ivec_n, 4 * (j - 2), st)
        out_v[pl.ds(g0, 16)] = res
        return carry

    lax.fori_loop(0, NGROUPS, group_body, 0)

    pltpu.sync_copy(out_v, out_hbm.at[pl.ds(base, BPW)])


def kernel(user_inputs, item_inputs, user_table, item_table):
    y = _sc_dot(user_inputs.astype(jnp.int32), item_inputs.astype(jnp.int32),
                user_table.T, item_table.T)
    return y.reshape(B, 1)


# speedup vs baseline: 1.1150x; 1.1150x over previous
import functools

import jax
import jax.numpy as jnp
from jax import lax
from jax.experimental import pallas as pl
from jax.experimental.pallas import tpu as pltpu
from jax.experimental.pallas import tpu_sc as plsc

B = 16384
D = 32
NUM_CORES = 2
NUM_SUBCORES = 16
NW = NUM_CORES * NUM_SUBCORES
BPW = B // NW
CHUNK = 2
NSETS = 4
NGROUPS = BPW // 16


@functools.partial(
    pl.kernel,
    out_type=jax.ShapeDtypeStruct((B,), jnp.float32),
    mesh=plsc.VectorSubcoreMesh(core_axis_name="c", subcore_axis_name="s"),
    scratch_types=[
        pltpu.VMEM((BPW + 16,), jnp.int32),
        pltpu.VMEM((BPW + 16,), jnp.int32),
        pltpu.VMEM((NSETS, CHUNK, D, 128), jnp.float32),
        pltpu.VMEM((NSETS, CHUNK, D, 128), jnp.float32),
        pltpu.VMEM((BPW,), jnp.float32),
        pltpu.SemaphoreType.DMA,
        pltpu.SemaphoreType.DMA,
        pltpu.SemaphoreType.DMA,
        pltpu.SemaphoreType.DMA,
    ],
    compiler_params=pltpu.CompilerParams(needs_layout_passes=False),
)
def _sc_dot(uidx_hbm, iidx_hbm, utab_t_hbm, itab_t_hbm, out_hbm,
            uidx_v, iidx_v, ublk_v, iblk_v, out_v,
            semh0, semh1, semh2, semh3):
    wid = lax.axis_index("s") * NUM_CORES + lax.axis_index("c")
    base = wid * BPW
    semh = (semh0, semh1, semh2, semh3)

    pltpu.sync_copy(uidx_hbm.at[pl.ds(base, BPW)], uidx_v.at[pl.ds(0, BPW)])
    pltpu.sync_copy(iidx_hbm.at[pl.ds(base, BPW)], iidx_v.at[pl.ds(0, BPW)])

    lanes16 = lax.iota(jnp.int32, 16)
    d_lo = lanes16
    d_hi = lanes16 + 16

    def issue_hbm(uvec, ivec, k0, st):
        for i in range(CHUNK):
            r0u = pl.multiple_of((uvec[k0 + i] // 128) * 128, 128)
            r0i = pl.multiple_of((ivec[k0 + i] // 128) * 128, 128)
            pltpu.async_copy(utab_t_hbm.at[:, pl.ds(r0u, 128)],
                             ublk_v.at[st, i], semh[st])
            pltpu.async_copy(itab_t_hbm.at[:, pl.ds(r0i, 128)],
                             iblk_v.at[st, i], semh[st])

    def drain_hbm(st):
        for i in range(CHUNK):
            pltpu.make_async_copy(utab_t_hbm.at[:, pl.ds(0, 128)],
                                  ublk_v.at[st, i], semh[st]).wait()
            pltpu.make_async_copy(utab_t_hbm.at[:, pl.ds(0, 128)],
                                  iblk_v.at[st, i], semh[st]).wait()

    uvec0 = uidx_v[pl.ds(0, 16)]
    ivec0 = iidx_v[pl.ds(0, 16)]
    for j in range(NSETS):
        issue_hbm(uvec0, ivec0, 2 * j, j)

    def group_body(q, carry):
        g0 = q * 16
        uvec = uidx_v[pl.ds(g0, 16)]
        ivec = iidx_v[pl.ds(g0, 16)]
        uvec_n = uidx_v[pl.ds(g0 + 16, 16)]
        ivec_n = iidx_v[pl.ds(g0 + 16, 16)]
        res = jnp.zeros((16,), jnp.float32)
        for j in range(8):
            st = j % NSETS
            drain_hbm(st)
            for i in range(CHUNK):
                k = 2 * j + i
                lu = jnp.full((16,), uvec[k] % 128, jnp.int32)
                li = jnp.full((16,), ivec[k] % 128, jnp.int32)
                st_s = jnp.full((16,), st, jnp.int32)
                i_s = jnp.full((16,), i, jnp.int32)
                u1 = plsc.load_gather(ublk_v, [st_s, i_s, d_lo, lu])
                u2 = plsc.load_gather(ublk_v, [st_s, i_s, d_hi, lu])
                v1 = plsc.load_gather(iblk_v, [st_s, i_s, d_lo, li])
                v2 = plsc.load_gather(iblk_v, [st_s, i_s, d_hi, li])
                dot = jnp.sum(u1 * v1 + u2 * v2)
                res = jnp.where(lanes16 == k, dot, res)
            if j < 4:
                issue_hbm(uvec, ivec, 2 * (j + 4), st)
            else:
                @pl.when(q < NGROUPS - 1)
                def _():
                    issue_hbm(uvec_n, ivec_n, 2 * (j - 4), st)
        out_v[pl.ds(g0, 16)] = res
        return carry

    lax.fori_loop(0, NGROUPS, group_body, 0)

    pltpu.sync_copy(out_v, out_hbm.at[pl.ds(base, BPW)])


def kernel(user_inputs, item_inputs, user_table, item_table):
    y = _sc_dot(user_inputs.astype(jnp.int32), item_inputs.astype(jnp.int32),
                user_table.T, item_table.T)
    return y.reshape(B, 1)


# speedup vs baseline: 1.1943x; 1.0711x over previous
import functools

import jax
import jax.numpy as jnp
from jax import lax
from jax.experimental import pallas as pl
from jax.experimental.pallas import tpu as pltpu
from jax.experimental.pallas import tpu_sc as plsc

B = 16384
D = 32
NUM_CORES = 2
NUM_SUBCORES = 16
NW = NUM_CORES * NUM_SUBCORES
BPW = B // NW
CHUNK = 1
NSETS = 8
NGROUPS = BPW // 16


@functools.partial(
    pl.kernel,
    out_type=jax.ShapeDtypeStruct((B,), jnp.float32),
    mesh=plsc.VectorSubcoreMesh(core_axis_name="c", subcore_axis_name="s"),
    scratch_types=[
        pltpu.VMEM((BPW + 16,), jnp.int32),
        pltpu.VMEM((BPW + 16,), jnp.int32),
        pltpu.VMEM((NSETS, CHUNK, D, 128), jnp.float32),
        pltpu.VMEM((NSETS, CHUNK, D, 128), jnp.float32),
        pltpu.VMEM((BPW,), jnp.float32),
        pltpu.SemaphoreType.DMA,
        pltpu.SemaphoreType.DMA,
        pltpu.SemaphoreType.DMA,
        pltpu.SemaphoreType.DMA,
        pltpu.SemaphoreType.DMA,
        pltpu.SemaphoreType.DMA,
        pltpu.SemaphoreType.DMA,
        pltpu.SemaphoreType.DMA,
    ],
    compiler_params=pltpu.CompilerParams(needs_layout_passes=False),
)
def _sc_dot(uidx_hbm, iidx_hbm, utab_t_hbm, itab_t_hbm, out_hbm,
            uidx_v, iidx_v, ublk_v, iblk_v, out_v,
            semh0, semh1, semh2, semh3, semh4, semh5, semh6, semh7):
    wid = lax.axis_index("s") * NUM_CORES + lax.axis_index("c")
    base = wid * BPW
    semh = (semh0, semh1, semh2, semh3, semh4, semh5, semh6, semh7)

    pltpu.sync_copy(uidx_hbm.at[pl.ds(base, BPW)], uidx_v.at[pl.ds(0, BPW)])
    pltpu.sync_copy(iidx_hbm.at[pl.ds(base, BPW)], iidx_v.at[pl.ds(0, BPW)])

    lanes16 = lax.iota(jnp.int32, 16)
    d_lo = lanes16
    d_hi = lanes16 + 16

    def issue_hbm(uvec, ivec, k0, st):
        for i in range(CHUNK):
            r0u = pl.multiple_of((uvec[k0 + i] // 128) * 128, 128)
            r0i = pl.multiple_of((ivec[k0 + i] // 128) * 128, 128)
            pltpu.async_copy(utab_t_hbm.at[:, pl.ds(r0u, 128)],
                             ublk_v.at[st, i], semh[st])
            pltpu.async_copy(itab_t_hbm.at[:, pl.ds(r0i, 128)],
                             iblk_v.at[st, i], semh[st])

    def drain_hbm(st):
        for i in range(CHUNK):
            pltpu.make_async_copy(utab_t_hbm.at[:, pl.ds(0, 128)],
                                  ublk_v.at[st, i], semh[st]).wait()
            pltpu.make_async_copy(utab_t_hbm.at[:, pl.ds(0, 128)],
                                  iblk_v.at[st, i], semh[st]).wait()

    uvec0 = uidx_v[pl.ds(0, 16)]
    ivec0 = iidx_v[pl.ds(0, 16)]
    for j in range(NSETS):
        issue_hbm(uvec0, ivec0, j, j)

    def group_body(q, carry):
        g0 = q * 16
        uvec = uidx_v[pl.ds(g0, 16)]
        ivec = iidx_v[pl.ds(g0, 16)]
        uvec_n = uidx_v[pl.ds(g0 + 16, 16)]
        ivec_n = iidx_v[pl.ds(g0 + 16, 16)]
        res = jnp.zeros((16,), jnp.float32)
        for j in range(16):
            st = j % NSETS
            drain_hbm(st)
            for i in range(CHUNK):
                k = j + i
                lu = jnp.full((16,), uvec[k] % 128, jnp.int32)
                li = jnp.full((16,), ivec[k] % 128, jnp.int32)
                st_s = jnp.full((16,), st, jnp.int32)
                i_s = jnp.full((16,), i, jnp.int32)
                u1 = plsc.load_gather(ublk_v, [st_s, i_s, d_lo, lu])
                u2 = plsc.load_gather(ublk_v, [st_s, i_s, d_hi, lu])
                v1 = plsc.load_gather(iblk_v, [st_s, i_s, d_lo, li])
                v2 = plsc.load_gather(iblk_v, [st_s, i_s, d_hi, li])
                dot = jnp.sum(u1 * v1 + u2 * v2)
                res = jnp.where(lanes16 == k, dot, res)
            if j < 8:
                issue_hbm(uvec, ivec, j + 8, st)
            else:
                @pl.when(q < NGROUPS - 1)
                def _():
                    issue_hbm(uvec_n, ivec_n, j - 8, st)
        out_v[pl.ds(g0, 16)] = res
        return carry

    lax.fori_loop(0, NGROUPS, group_body, 0)

    pltpu.sync_copy(out_v, out_hbm.at[pl.ds(base, BPW)])


def kernel(user_inputs, item_inputs, user_table, item_table):
    y = _sc_dot(user_inputs.astype(jnp.int32), item_inputs.astype(jnp.int32),
                user_table.T, item_table.T)
    return y.reshape(B, 1)
